# baseline (speedup 1.0000x reference)
import functools

import jax
import jax.numpy as jnp
from jax import lax
from jax.experimental import pallas as pl
from jax.experimental.pallas import tpu as pltpu
from jax.experimental.pallas import tpu_sc as plsc

B = 16384
C = 1000
L = 16
NC = 2
NS = 16
NW = NC * NS
PER_W = B // NW
CHUNKS = PER_W // L

_SIGMA = 0.5
_SCALE = -1.0 / B


@functools.partial(
    pl.kernel,
    out_type=jax.ShapeDtypeStruct((NW, L), jnp.float32),
    mesh=plsc.VectorSubcoreMesh(core_axis_name="c", subcore_axis_name="s"),
    scratch_types=[
        pltpu.VMEM((PER_W,), jnp.int32),
        pltpu.VMEM((PER_W,), jnp.float32),
        pltpu.VMEM((PER_W,), jnp.int32),
        pltpu.VMEM((PER_W,), jnp.float32),
        pltpu.VMEM((L,), jnp.float32),
        pltpu.SemaphoreType.DMA,
    ],
)
def _sc_loss(preds_hbm, targets_hbm, margin_hbm, out_hbm,
             t_v, m_v, idx_v, n_v, res_v, sem):
    wid = lax.axis_index("s") * NC + lax.axis_index("c")
    base = wid * PER_W

    pltpu.sync_copy(targets_hbm.at[pl.ds(base, PER_W)], t_v)
    pltpu.sync_copy(margin_hbm.at[pl.ds(base, PER_W)], m_v)

    iota = lax.iota(jnp.int32, L)
    for j in range(CHUNKS):
        i_vec = iota + (base + j * L)
        idx_v[pl.ds(j * L, L)] = i_vec * C + t_v[pl.ds(j * L, L)]

    pltpu.async_copy(preds_hbm.at[idx_v], n_v, sem).wait()

    acc = jnp.zeros((L,), jnp.float32)
    for j in range(CHUNKS):
        v = n_v[pl.ds(j * L, L)]
        m = m_v[pl.ds(j * L, L)]
        w = jnp.exp(-_SIGMA * m * m)
        nz = (m > 0.0) | (m < 0.0)
        acc = acc + jnp.where(nz, w * v, 0.0)

    res_v[...] = acc * _SCALE
    pltpu.sync_copy(res_v, out_hbm.at[wid])


def kernel(preds, targets, margin):
    preds_flat = preds.reshape(-1)
    partials = _sc_loss(preds_flat, targets.astype(jnp.int32), margin)
    return jnp.sum(partials)


# speedup vs baseline: 1.2423x; 1.2423x over previous
import functools

import jax
import jax.numpy as jnp
from jax import lax
from jax.experimental import pallas as pl
from jax.experimental.pallas import tpu as pltpu

B = 16384
C = 1000
BR = 256
NB = B // BR

_SIGMA = 0.5
_SCALE = -1.0 / B


def _tc_body(p_ref, t_ref, m_ref, out_ref):
    a = pl.program_id(0)

    @pl.when(a == 0)
    def _():
        out_ref[...] = jnp.zeros_like(out_ref)

    t_row = t_ref[0]
    m_row = m_ref[0]
    w_row = jnp.exp(-_SIGMA * m_row * m_row)
    w_row = jnp.where((m_row > 0.0) | (m_row < 0.0), w_row, 0.0)
    r_i = lax.broadcasted_iota(jnp.int32, (BR, BR), 0)
    c_i = lax.broadcasted_iota(jnp.int32, (BR, BR), 1)
    ident = jnp.where(r_i == c_i, 1.0, 0.0).astype(jnp.float32)
    both = jnp.concatenate([t_row, w_row], axis=0)
    cols_sb = lax.dot_general(
        ident, both, (((1,), (1,)), ((), ())),
        precision=lax.Precision.HIGHEST,
        preferred_element_type=jnp.float32)
    t_col = cols_sb[:, 0:1].astype(jnp.int32)
    w_col = cols_sb[:, 1:2]

    p = p_ref[...]
    t_b = jnp.broadcast_to(t_col, (BR, 128))
    w_b = jnp.broadcast_to(w_col, (BR, 128))
    tile = lax.broadcasted_iota(jnp.int32, (BR, 128), 1)
    acc = jnp.zeros((BR, 128), jnp.float32)
    for k in range(C // 128):
        lo = k * 128
        acc = acc + jnp.where(tile + lo == t_b, p[:, lo:lo + 128], 0.0)
    lo = (C // 128) * 128
    tail_w = C - lo
    tail = jnp.where(tile[:, :tail_w] + lo == t_b[:, :tail_w], p[:, lo:], 0.0)
    tail = jnp.concatenate(
        [tail, jnp.zeros((BR, 128 - tail_w), jnp.float32)], axis=1)
    acc = acc + tail
    out_ref[...] += acc * w_b


_tc_reduce = pl.pallas_call(
    _tc_body,
    grid=(NB,),
    in_specs=[
        pl.BlockSpec((BR, C), lambda a: (a, 0)),
        pl.BlockSpec((1, 1, BR), lambda a: (a, 0, 0)),
        pl.BlockSpec((1, 1, BR), lambda a: (a, 0, 0)),
    ],
    out_specs=pl.BlockSpec((BR, 128), lambda a: (0, 0)),
    out_shape=jax.ShapeDtypeStruct((BR, 128), jnp.float32),
)


def kernel(preds, targets, margin):
    targets_l = targets.astype(jnp.float32).reshape(NB, 1, BR)
    margin_l = margin.reshape(NB, 1, BR)
    partials = _tc_reduce(preds, targets_l, margin_l)
    return jnp.sum(partials) * _SCALE


# speedup vs baseline: 5.8927x; 4.7435x over previous
import jax
import jax.numpy as jnp
from jax import lax
from jax.experimental import pallas as pl
from jax.experimental.pallas import tpu as pltpu

B = 16384
C = 1000
BC = 2048
NBLK = B // BC
RCH = 8

_SIGMA = 0.5
_SCALE = -1.0 / B


def _tc_body(p_ref, t_ref, m_ref, out_ref):
    b = pl.program_id(0)

    @pl.when(b == 0)
    def _():
        out_ref[...] = jnp.zeros_like(out_ref)

    t = t_ref[0]
    m = m_ref[0]
    w = jnp.exp(-_SIGMA * m * m)
    w = jnp.where((m > 0.0) | (m < 0.0), w, 0.0)

    ri = lax.broadcasted_iota(jnp.int32, (RCH, BC), 0)
    acc = jnp.zeros((RCH, BC), jnp.float32)
    for k in range(C // RCH):
        pk = p_ref[pl.ds(k * RCH, RCH), :]
        acc = acc + jnp.where(ri == t - k * RCH, pk, 0.0)
    out_ref[...] += acc * w


_tc_reduce = pl.pallas_call(
    _tc_body,
    grid=(NBLK,),
    in_specs=[
        pl.BlockSpec((C, BC), lambda b: (0, b)),
        pl.BlockSpec((1, 1, BC), lambda b: (b, 0, 0)),
        pl.BlockSpec((1, 1, BC), lambda b: (b, 0, 0)),
    ],
    out_specs=pl.BlockSpec((RCH, BC), lambda b: (0, 0)),
    out_shape=jax.ShapeDtypeStruct((RCH, BC), jnp.float32),
)


def kernel(preds, targets, margin):
    preds_t = preds.T
    t3 = targets.astype(jnp.int32).reshape(NBLK, 1, BC)
    m3 = margin.reshape(NBLK, 1, BC)
    partials = _tc_reduce(preds_t, t3, m3)
    return jnp.sum(partials) * _SCALE
